# Initial kernel scaffold; baseline (speedup 1.0000x reference)
import jax
import jax.numpy as jnp
from jax.experimental import pallas as pl


def kernel(x, edge_index, batch, W_gcn, b_gcn, enc_W1, enc_b1, enc_W2, enc_b2, enc_W3, enc_b3, dec_W1, dec_b1, dec_W2, dec_b2, dec_W3, dec_b3):
    raise NotImplementedError("write your pallas kernel here")


# speedup vs baseline: 10.4901x; 10.4901x over previous
import functools

import jax
import jax.numpy as jnp
from jax import lax
from jax.experimental import pallas as pl
from jax.experimental.pallas import tpu as pltpu
from jax.experimental.pallas import tpu_sc as plsc

N = 10000
E = 320000
D = 128
G = 64
H_MLP = 64
D_OUT = 64

NC, NS, L = 2, 16, 16
NW = NC * NS
CHUNK = 128
EPW = 10240
EPAD = NW * EPW
NCH = EPW // CHUNK
NPAD = 10240
RPT = NPAD // NS

_sc_mesh = plsc.VectorSubcoreMesh(
    core_axis_name="c", subcore_axis_name="s", num_cores=NC, num_subcores=NS)


@functools.partial(
    pl.kernel,
    out_type=jax.ShapeDtypeStruct((NC, NPAD), jnp.float32),
    mesh=_sc_mesh,
    scratch_types=[
        pltpu.VMEM((NCH, CHUNK), jnp.int32),
        pltpu.VMEM((CHUNK,), jnp.float32),
        pltpu.VMEM((RPT,), jnp.float32),
        pltpu.VMEM_SHARED((NPAD,), jnp.float32),
    ],
)
def _deg_kernel(dst_hbm, zeros_hbm, out_hbm, idx_v, ones_v, buf_v, hist_sh):
    c = lax.axis_index("c")
    s = lax.axis_index("s")
    w = c * NS + s
    for k in range(CHUNK // L):
        ones_v[pl.ds(k * L, L)] = jnp.ones((L,), jnp.float32)
    pltpu.sync_copy(zeros_hbm, buf_v)
    pltpu.sync_copy(buf_v, hist_sh.at[pl.ds(s * RPT, RPT)])
    pltpu.sync_copy(dst_hbm.at[pl.ds(w * NCH, NCH)], idx_v)
    plsc.subcore_barrier()

    @pl.loop(0, NCH)
    def _(j):
        pltpu.sync_copy(ones_v, hist_sh.at[idx_v.at[j]], add=True)

    plsc.subcore_barrier()
    pltpu.sync_copy(hist_sh.at[pl.ds(s * RPT, RPT)], buf_v)
    pltpu.sync_copy(buf_v, out_hbm.at[c].at[pl.ds(s * RPT, RPT)])


RB = 1000


def _mm_body(x_ref, w_ref, hist_ref, h2_ref, dinv_ref):
    deg = hist_ref[0] + hist_ref[1] + 1.0
    dinv = lax.rsqrt(deg)
    h = jnp.dot(x_ref[...], w_ref[...], preferred_element_type=jnp.float32,
                precision=lax.Precision.HIGHEST)
    h2_ref[...] = h * dinv
    dinv_ref[...] = dinv


_mm_call = pl.pallas_call(
    _mm_body,
    grid=(N // RB,),
    in_specs=[
        pl.BlockSpec((RB, D), lambda i: (i, 0)),
        pl.BlockSpec((D, D), lambda i: (0, 0)),
        pl.BlockSpec((NC, RB, 1), lambda i: (0, i, 0)),
    ],
    out_specs=[
        pl.BlockSpec((RB, D), lambda i: (i, 0)),
        pl.BlockSpec((RB, 1), lambda i: (i, 0)),
    ],
    out_shape=[
        jax.ShapeDtypeStruct((N, D), jnp.float32),
        jax.ShapeDtypeStruct((N, 1), jnp.float32),
    ],
)


@functools.partial(
    pl.kernel,
    out_type=jax.ShapeDtypeStruct((NC, NPAD, D), jnp.float32),
    mesh=_sc_mesh,
    scratch_types=[
        pltpu.VMEM((NCH, CHUNK), jnp.int32),
        pltpu.VMEM((NCH, CHUNK), jnp.int32),
        pltpu.VMEM((CHUNK, D), jnp.float32),
        pltpu.SemaphoreType.DMA,
        pltpu.VMEM_SHARED((NPAD, D), jnp.float32),
    ],
)
def _agg_kernel(h2_hbm, src_hbm, dst_hbm, zeros_hbm, out_hbm,
                sidx_v, didx_v, buf_v, sem, acc_sh):
    c = lax.axis_index("c")
    s = lax.axis_index("s")
    w = c * NS + s
    pltpu.sync_copy(zeros_hbm, buf_v)
    for k in range(RPT // CHUNK):
        pltpu.sync_copy(buf_v, acc_sh.at[pl.ds(s * RPT + k * CHUNK, CHUNK)])
    pltpu.sync_copy(src_hbm.at[pl.ds(w * NCH, NCH)], sidx_v)
    pltpu.sync_copy(dst_hbm.at[pl.ds(w * NCH, NCH)], didx_v)
    plsc.subcore_barrier()

    @pl.loop(0, NCH)
    def _(j):
        pltpu.async_copy(h2_hbm.at[sidx_v.at[j]], buf_v, sem).wait()
        pltpu.sync_copy(buf_v, acc_sh.at[didx_v.at[j]], add=True)

    plsc.subcore_barrier()
    for k in range(RPT // CHUNK):
        r = s * RPT + k * CHUNK
        pltpu.sync_copy(acc_sh.at[pl.ds(r, CHUNK)], buf_v)
        pltpu.sync_copy(buf_v, out_hbm.at[c].at[pl.ds(r, CHUNK)])


def _dot(a, b):
    return jnp.dot(a, b, preferred_element_type=jnp.float32,
                   precision=lax.Precision.HIGHEST)


def _head_body(p_ref, h2_ref, dinv_ref, bg_ref, bcol_ref, brow_ref,
               ew1_ref, eb1_ref, ew2_ref, eb2_ref, ew3_ref, eb3_ref,
               dw1_ref, db1_ref, dw2_ref, db2_ref, dw3_ref, db3_ref,
               eps_ref, mu_ref, std_ref, y_ref, gmax_scr):
    agg = p_ref[0, :N, :] + p_ref[1, :N, :]
    nx = jnp.maximum((agg + h2_ref[...]) * dinv_ref[...] + bg_ref[...], 0.0)
    iota_g = lax.broadcasted_iota(jnp.int32, (G, N), 0)
    oh = (brow_ref[...] == iota_g).astype(jnp.float32)
    gsum = _dot(oh, nx)
    cnt = jnp.sum(oh, axis=1, keepdims=True)
    gmean = gsum / jnp.maximum(cnt, 1.0)
    bcol = bcol_ref[...]

    def body(g, carry):
        m = jnp.where(bcol == g, nx, 0.0)
        row = jnp.max(m, axis=0, keepdims=True)
        gmax_scr[pl.ds(g, 1), :] = row
        return carry

    lax.fori_loop(0, G, body, 0)
    gmax = gmax_scr[...]
    gx = jnp.concatenate([gmean, gmax], axis=1)

    def elu(v):
        return jnp.where(v > 0, v, jnp.exp(v) - 1.0)

    h1 = elu(_dot(gx, ew1_ref[...]) + eb1_ref[...])
    h2m = jnp.tanh(_dot(h1, ew2_ref[...]) + eb2_ref[...])
    enc = _dot(h2m, ew3_ref[...]) + eb3_ref[...]
    mu = enc[:, :D_OUT]
    logvar = enc[:, D_OUT:]
    softplus = jnp.maximum(logvar, 0.0) + jnp.log1p(jnp.exp(-jnp.abs(logvar)))
    std = 1e-6 + softplus
    z = mu + eps_ref[...] * std
    d1 = jnp.tanh(_dot(z, dw1_ref[...]) + db1_ref[...])
    d2 = elu(_dot(d1, dw2_ref[...]) + db2_ref[...])
    y = 1.0 / (1.0 + jnp.exp(-(_dot(d2, dw3_ref[...]) + db3_ref[...])))
    mu_ref[...] = mu
    std_ref[...] = std
    y_ref[...] = jnp.clip(y, 1e-8, 1.0 - 1e-8)


_head_call = pl.pallas_call(
    _head_body,
    out_shape=[
        jax.ShapeDtypeStruct((G, D_OUT), jnp.float32),
        jax.ShapeDtypeStruct((G, D_OUT), jnp.float32),
        jax.ShapeDtypeStruct((G, D), jnp.float32),
    ],
    scratch_shapes=[pltpu.VMEM((G, D), jnp.float32)],
)


def kernel(x, edge_index, batch, W_gcn, b_gcn, enc_W1, enc_b1, enc_W2, enc_b2,
           enc_W3, enc_b3, dec_W1, dec_b1, dec_W2, dec_b2, dec_W3, dec_b3):
    pad = EPAD - E
    src_p = jnp.concatenate(
        [edge_index[0], jnp.zeros((pad,), jnp.int32)]).reshape(EPAD // CHUNK, CHUNK)
    pad_dst = N + (jnp.arange(pad, dtype=jnp.int32) % (NPAD - N))
    dst_p = jnp.concatenate([edge_index[1], pad_dst]).reshape(EPAD // CHUNK, CHUNK)
    zeros1 = jnp.zeros((RPT,), jnp.float32)
    zeros2 = jnp.zeros((CHUNK, D), jnp.float32)

    hist = _deg_kernel(dst_p, zeros1)
    h2, dinv = _mm_call(x, W_gcn, hist.reshape(NC, NPAD, 1))
    p = _agg_kernel(h2, src_p, dst_p, zeros2)

    eps = jax.random.normal(jax.random.key(42), (G, D_OUT), dtype=jnp.float32)
    mu, std, y = _head_call(
        p, h2, dinv, b_gcn.reshape(1, D), batch.reshape(N, 1), batch.reshape(1, N),
        enc_W1, enc_b1.reshape(1, H_MLP), enc_W2, enc_b2.reshape(1, H_MLP),
        enc_W3, enc_b3.reshape(1, 2 * D_OUT), dec_W1, dec_b1.reshape(1, H_MLP),
        dec_W2, dec_b2.reshape(1, H_MLP), dec_W3, dec_b3.reshape(1, D), eps)
    return (mu, std, y)


# speedup vs baseline: 11.8488x; 1.1295x over previous
import functools

import jax
import jax.numpy as jnp
from jax import lax
from jax.experimental import pallas as pl
from jax.experimental.pallas import tpu as pltpu
from jax.experimental.pallas import tpu_sc as plsc

N = 10000
E = 320000
D = 128
G = 64
H_MLP = 64
D_OUT = 64

NC, NS, L = 2, 16, 16
NW = NC * NS
CHUNK = 128
EPW = 10240
EPAD = NW * EPW
NCH = EPW // CHUNK
NPAD = 10240
RPT = NPAD // NS
NBUF = 2

_sc_mesh = plsc.VectorSubcoreMesh(
    core_axis_name="c", subcore_axis_name="s", num_cores=NC, num_subcores=NS)


@functools.partial(
    pl.kernel,
    out_type=jax.ShapeDtypeStruct((NC, NPAD), jnp.float32),
    mesh=_sc_mesh,
    scratch_types=[
        pltpu.VMEM((NCH, CHUNK), jnp.int32),
        pltpu.VMEM((CHUNK,), jnp.float32),
        pltpu.VMEM((RPT,), jnp.float32),
        pltpu.VMEM_SHARED((NPAD,), jnp.float32),
    ],
)
def _deg_kernel(dst_hbm, zeros_hbm, out_hbm, idx_v, ones_v, buf_v, hist_sh):
    c = lax.axis_index("c")
    s = lax.axis_index("s")
    w = c * NS + s
    for k in range(CHUNK // L):
        ones_v[pl.ds(k * L, L)] = jnp.ones((L,), jnp.float32)
    pltpu.sync_copy(zeros_hbm, buf_v)
    pltpu.sync_copy(buf_v, hist_sh.at[pl.ds(s * RPT, RPT)])
    pltpu.sync_copy(dst_hbm.at[pl.ds(w * NCH, NCH)], idx_v)
    plsc.subcore_barrier()

    @pl.loop(0, NCH)
    def _(j):
        pltpu.sync_copy(ones_v, hist_sh.at[idx_v.at[j]], add=True)

    plsc.subcore_barrier()
    pltpu.sync_copy(hist_sh.at[pl.ds(s * RPT, RPT)], buf_v)
    pltpu.sync_copy(buf_v, out_hbm.at[c].at[pl.ds(s * RPT, RPT)])


RB = 1000


def _mm_body(x_ref, w_ref, hist_ref, h2_ref, dinv_ref):
    deg = hist_ref[0] + hist_ref[1] + 1.0
    dinv = lax.rsqrt(deg)
    h = jnp.dot(x_ref[...], w_ref[...], preferred_element_type=jnp.float32,
                precision=lax.Precision.HIGHEST)
    h2_ref[...] = h * dinv
    dinv_ref[...] = dinv


_mm_call = pl.pallas_call(
    _mm_body,
    grid=(N // RB,),
    in_specs=[
        pl.BlockSpec((RB, D), lambda i: (i, 0)),
        pl.BlockSpec((D, D), lambda i: (0, 0)),
        pl.BlockSpec((NC, RB, 1), lambda i: (0, i, 0)),
    ],
    out_specs=[
        pl.BlockSpec((RB, D), lambda i: (i, 0)),
        pl.BlockSpec((RB, 1), lambda i: (i, 0)),
    ],
    out_shape=[
        jax.ShapeDtypeStruct((N, D), jnp.float32),
        jax.ShapeDtypeStruct((N, 1), jnp.float32),
    ],
)


NCH0 = 120
NCH1 = 40
STAGE = 40


@functools.partial(
    pl.kernel,
    out_type=jax.ShapeDtypeStruct((NC, NPAD, D), jnp.float32),
    mesh=_sc_mesh,
    scratch_types=[
        pltpu.VMEM((STAGE, CHUNK), jnp.int32),
        pltpu.VMEM((STAGE, CHUNK), jnp.int32),
        pltpu.VMEM((NBUF, CHUNK, D), jnp.float32),
        pltpu.SemaphoreType.DMA,
        pltpu.SemaphoreType.DMA,
        pltpu.VMEM_SHARED((NPAD, D), jnp.float32),
    ],
)
def _agg_kernel(h2_hbm, src_hbm, dst_hbm, zeros_hbm, out_hbm,
                sidx_v, didx_v, bufs_v, sem0, sem1, acc_sh):
    c = lax.axis_index("c")
    s = lax.axis_index("s")
    sems = [sem0, sem1]
    pltpu.sync_copy(zeros_hbm, bufs_v.at[0])
    for k in range(RPT // CHUNK):
        pltpu.sync_copy(bufs_v.at[0], acc_sh.at[pl.ds(s * RPT + k * CHUNK, CHUNK)])

    def run_edges(row0, nch):
        for st in range(nch // STAGE):
            r0 = row0 + st * STAGE
            pltpu.sync_copy(src_hbm.at[pl.ds(r0, STAGE)], sidx_v)
            pltpu.sync_copy(dst_hbm.at[pl.ds(r0, STAGE)], didx_v)

            @pl.loop(0, STAGE // NBUF)
            def _(g):
                base = g * NBUF
                cps = [pltpu.async_copy(h2_hbm.at[sidx_v.at[base + b]],
                                        bufs_v.at[b], sems[b])
                       for b in range(NBUF)]
                for b in range(NBUF):
                    cps[b].wait()
                    pltpu.sync_copy(bufs_v.at[b], acc_sh.at[didx_v.at[base + b]],
                                    add=True)

    plsc.subcore_barrier()

    @pl.when(c == 0)
    def _():
        run_edges(s * NCH0, NCH0)

    @pl.when(c == 1)
    def _():
        run_edges(NS * NCH0 + s * NCH1, NCH1)

    plsc.subcore_barrier()
    for k in range(RPT // CHUNK):
        r = s * RPT + k * CHUNK
        pltpu.sync_copy(acc_sh.at[pl.ds(r, CHUNK)], bufs_v.at[0])
        pltpu.sync_copy(bufs_v.at[0], out_hbm.at[c].at[pl.ds(r, CHUNK)])


def _dot(a, b):
    return jnp.dot(a, b, preferred_element_type=jnp.float32,
                   precision=lax.Precision.HIGHEST)


def _head_body(p_ref, h2_ref, dinv_ref, bg_ref, bcol_ref, brow_ref,
               ew1_ref, eb1_ref, ew2_ref, eb2_ref, ew3_ref, eb3_ref,
               dw1_ref, db1_ref, dw2_ref, db2_ref, dw3_ref, db3_ref,
               eps_ref, mu_ref, std_ref, y_ref, gmax_scr):
    agg = p_ref[0, :N, :] + p_ref[1, :N, :]
    nx = jnp.maximum((agg + h2_ref[...]) * dinv_ref[...] + bg_ref[...], 0.0)
    iota_g = lax.broadcasted_iota(jnp.int32, (G, N), 0)
    oh = (brow_ref[...] == iota_g).astype(jnp.float32)
    gsum = _dot(oh, nx)
    cnt = jnp.sum(oh, axis=1, keepdims=True)
    gmean = gsum / jnp.maximum(cnt, 1.0)
    bcol = bcol_ref[...]

    def body(g, carry):
        m = jnp.where(bcol == g, nx, 0.0)
        row = jnp.max(m, axis=0, keepdims=True)
        gmax_scr[pl.ds(g, 1), :] = row
        return carry

    lax.fori_loop(0, G, body, 0)
    gmax = gmax_scr[...]
    gx = jnp.concatenate([gmean, gmax], axis=1)

    def elu(v):
        return jnp.where(v > 0, v, jnp.exp(v) - 1.0)

    h1 = elu(_dot(gx, ew1_ref[...]) + eb1_ref[...])
    h2m = jnp.tanh(_dot(h1, ew2_ref[...]) + eb2_ref[...])
    enc = _dot(h2m, ew3_ref[...]) + eb3_ref[...]
    mu = enc[:, :D_OUT]
    logvar = enc[:, D_OUT:]
    softplus = jnp.maximum(logvar, 0.0) + jnp.log1p(jnp.exp(-jnp.abs(logvar)))
    std = 1e-6 + softplus
    z = mu + eps_ref[...] * std
    d1 = jnp.tanh(_dot(z, dw1_ref[...]) + db1_ref[...])
    d2 = elu(_dot(d1, dw2_ref[...]) + db2_ref[...])
    y = 1.0 / (1.0 + jnp.exp(-(_dot(d2, dw3_ref[...]) + db3_ref[...])))
    mu_ref[...] = mu
    std_ref[...] = std
    y_ref[...] = jnp.clip(y, 1e-8, 1.0 - 1e-8)


_head_call = pl.pallas_call(
    _head_body,
    out_shape=[
        jax.ShapeDtypeStruct((G, D_OUT), jnp.float32),
        jax.ShapeDtypeStruct((G, D_OUT), jnp.float32),
        jax.ShapeDtypeStruct((G, D), jnp.float32),
    ],
    scratch_shapes=[pltpu.VMEM((G, D), jnp.float32)],
)


def kernel(x, edge_index, batch, W_gcn, b_gcn, enc_W1, enc_b1, enc_W2, enc_b2,
           enc_W3, enc_b3, dec_W1, dec_b1, dec_W2, dec_b2, dec_W3, dec_b3):
    pad = EPAD - E
    src_p = jnp.concatenate(
        [edge_index[0], jnp.zeros((pad,), jnp.int32)]).reshape(EPAD // CHUNK, CHUNK)
    pad_dst = N + (jnp.arange(pad, dtype=jnp.int32) % (NPAD - N))
    dst_p = jnp.concatenate([edge_index[1], pad_dst]).reshape(EPAD // CHUNK, CHUNK)
    zeros1 = jnp.zeros((RPT,), jnp.float32)
    zeros2 = jnp.zeros((CHUNK, D), jnp.float32)

    hist = _deg_kernel(dst_p, zeros1)
    h2, dinv = _mm_call(x, W_gcn, hist.reshape(NC, NPAD, 1))
    p = _agg_kernel(h2, src_p, dst_p, zeros2)

    eps = jax.random.normal(jax.random.key(42), (G, D_OUT), dtype=jnp.float32)
    mu, std, y = _head_call(
        p, h2, dinv, b_gcn.reshape(1, D), batch.reshape(N, 1), batch.reshape(1, N),
        enc_W1, enc_b1.reshape(1, H_MLP), enc_W2, enc_b2.reshape(1, H_MLP),
        enc_W3, enc_b3.reshape(1, 2 * D_OUT), dec_W1, dec_b1.reshape(1, H_MLP),
        dec_W2, dec_b2.reshape(1, H_MLP), dec_W3, dec_b3.reshape(1, D), eps)
    return (mu, std, y)


# speedup vs baseline: 13.3896x; 1.1300x over previous
import functools

import jax
import jax.numpy as jnp
from jax import lax
from jax.experimental import pallas as pl
from jax.experimental.pallas import tpu as pltpu
from jax.experimental.pallas import tpu_sc as plsc

N = 10000
E = 320000
D = 128
G = 64
H_MLP = 64
D_OUT = 64

NC, NS, L = 2, 16, 16
NW = NC * NS
CHUNK = 128
EPW = 10240
EPAD = NW * EPW
NCH = EPW // CHUNK
NPAD = 10240
RPT = NPAD // NS
NBUF = 2

_sc_mesh = plsc.VectorSubcoreMesh(
    core_axis_name="c", subcore_axis_name="s", num_cores=NC, num_subcores=NS)


@functools.partial(
    pl.kernel,
    out_type=jax.ShapeDtypeStruct((NC, NPAD), jnp.float32),
    mesh=_sc_mesh,
    scratch_types=[
        pltpu.VMEM((NCH, CHUNK), jnp.int32),
        pltpu.VMEM((CHUNK,), jnp.float32),
        pltpu.VMEM((RPT,), jnp.float32),
        pltpu.VMEM_SHARED((NPAD,), jnp.float32),
    ],
)
def _deg_kernel(dst_hbm, zeros_hbm, out_hbm, idx_v, ones_v, buf_v, hist_sh):
    c = lax.axis_index("c")
    s = lax.axis_index("s")
    w = c * NS + s
    for k in range(CHUNK // L):
        ones_v[pl.ds(k * L, L)] = jnp.ones((L,), jnp.float32)
    pltpu.sync_copy(zeros_hbm, buf_v)
    pltpu.sync_copy(buf_v, hist_sh.at[pl.ds(s * RPT, RPT)])
    pltpu.sync_copy(dst_hbm.at[pl.ds(w * NCH, NCH)], idx_v)
    plsc.subcore_barrier()

    @pl.loop(0, NCH)
    def _(j):
        pltpu.sync_copy(ones_v, hist_sh.at[idx_v.at[j]], add=True)

    plsc.subcore_barrier()
    pltpu.sync_copy(hist_sh.at[pl.ds(s * RPT, RPT)], buf_v)
    pltpu.sync_copy(buf_v, out_hbm.at[c].at[pl.ds(s * RPT, RPT)])


RB = 1000


def _mm_body(x_ref, w_ref, hist_ref, h2_ref, dinv_ref):
    deg = hist_ref[0] + hist_ref[1] + 1.0
    dinv = lax.rsqrt(deg)
    h = jnp.dot(x_ref[...], w_ref[...], preferred_element_type=jnp.float32,
                precision=lax.Precision.HIGHEST)
    h2_ref[...] = h * dinv
    dinv_ref[...] = dinv


_mm_call = pl.pallas_call(
    _mm_body,
    grid=(N // RB,),
    in_specs=[
        pl.BlockSpec((RB, D), lambda i: (i, 0)),
        pl.BlockSpec((D, D), lambda i: (0, 0)),
        pl.BlockSpec((NC, RB, 1), lambda i: (0, i, 0)),
    ],
    out_specs=[
        pl.BlockSpec((RB, D), lambda i: (i, 0)),
        pl.BlockSpec((RB, 1), lambda i: (i, 0)),
    ],
    out_shape=[
        jax.ShapeDtypeStruct((N, D), jnp.float32),
        jax.ShapeDtypeStruct((N, 1), jnp.float32),
    ],
)


NCH0 = 120
NCH1 = 40
STAGE = 40


@functools.partial(
    pl.kernel,
    out_type=jax.ShapeDtypeStruct((NC, NPAD, D), jnp.float32),
    mesh=_sc_mesh,
    scratch_types=[
        pltpu.VMEM((STAGE, CHUNK), jnp.int32),
        pltpu.VMEM((STAGE, CHUNK), jnp.int32),
        pltpu.VMEM((NBUF, CHUNK, D), jnp.float32),
        pltpu.SemaphoreType.DMA,
        pltpu.SemaphoreType.DMA,
        pltpu.VMEM_SHARED((NPAD, D), jnp.float32),
    ],
)
def _agg_kernel(h2_hbm, src_hbm, dst_hbm, zeros_hbm, out_hbm,
                sidx_v, didx_v, bufs_v, sem0, sem1, acc_sh):
    c = lax.axis_index("c")
    s = lax.axis_index("s")
    sems = [sem0, sem1]
    pltpu.sync_copy(zeros_hbm, bufs_v.at[0])
    for k in range(RPT // CHUNK):
        pltpu.sync_copy(bufs_v.at[0], acc_sh.at[pl.ds(s * RPT + k * CHUNK, CHUNK)])

    def run_edges(row0, nch):
        for st in range(nch // STAGE):
            r0 = row0 + st * STAGE
            pltpu.sync_copy(src_hbm.at[pl.ds(r0, STAGE)], sidx_v)
            pltpu.sync_copy(dst_hbm.at[pl.ds(r0, STAGE)], didx_v)

            @pl.loop(0, STAGE // NBUF)
            def _(g):
                base = g * NBUF
                cps = [pltpu.async_copy(h2_hbm.at[sidx_v.at[base + b]],
                                        bufs_v.at[b], sems[b])
                       for b in range(NBUF)]
                for b in range(NBUF):
                    cps[b].wait()
                    pltpu.sync_copy(bufs_v.at[b], acc_sh.at[didx_v.at[base + b]],
                                    add=True)

    plsc.subcore_barrier()

    @pl.when(c == 0)
    def _():
        run_edges(s * NCH0, NCH0)

    @pl.when(c == 1)
    def _():
        run_edges(NS * NCH0 + s * NCH1, NCH1)

    plsc.subcore_barrier()
    for k in range(RPT // CHUNK):
        r = s * RPT + k * CHUNK
        pltpu.sync_copy(acc_sh.at[pl.ds(r, CHUNK)], bufs_v.at[0])
        pltpu.sync_copy(bufs_v.at[0], out_hbm.at[c].at[pl.ds(r, CHUNK)])


def _dot(a, b):
    return jnp.dot(a, b, preferred_element_type=jnp.float32,
                   precision=lax.Precision.HIGHEST)


WIN = 400


def _head_body(p_ref, h2_ref, dinv_ref, bg_ref, bcol_ref, brow_ref, bnds_ref,
               ew1_ref, eb1_ref, ew2_ref, eb2_ref, ew3_ref, eb3_ref,
               dw1_ref, db1_ref, dw2_ref, db2_ref, dw3_ref, db3_ref,
               eps_ref, mu_ref, std_ref, y_ref, gmax_scr):
    agg = p_ref[0, :N, :] + p_ref[1, :N, :]
    nx = jnp.maximum((agg + h2_ref[...]) * dinv_ref[...] + bg_ref[...], 0.0)
    iota_g = lax.broadcasted_iota(jnp.int32, (G, N), 0)
    oh = (brow_ref[...] == iota_g).astype(jnp.float32)
    gsum = _dot(oh, nx)
    cnt = jnp.sum(oh, axis=1, keepdims=True)
    gmean = gsum / jnp.maximum(cnt, 1.0)
    bcol = bcol_ref[...]
    gmax_scr[...] = jnp.zeros((G, D), jnp.float32)
    for w in range(N // WIN):
        blk = nx[w * WIN:(w + 1) * WIN, :]
        bblk = bcol[w * WIN:(w + 1) * WIN, :]

        def body(g, carry, blk=blk, bblk=bblk):
            m = jnp.where(bblk == g, blk, 0.0)
            row = jnp.max(m, axis=0, keepdims=True)
            gmax_scr[pl.ds(g, 1), :] = jnp.maximum(gmax_scr[pl.ds(g, 1), :], row)
            return carry

        lax.fori_loop(bnds_ref[w, 0], bnds_ref[w, 1] + 1, body, 0)
    gmax = gmax_scr[...]
    gx = jnp.concatenate([gmean, gmax], axis=1)

    def elu(v):
        return jnp.where(v > 0, v, jnp.exp(v) - 1.0)

    h1 = elu(_dot(gx, ew1_ref[...]) + eb1_ref[...])
    h2m = jnp.tanh(_dot(h1, ew2_ref[...]) + eb2_ref[...])
    enc = _dot(h2m, ew3_ref[...]) + eb3_ref[...]
    mu = enc[:, :D_OUT]
    logvar = enc[:, D_OUT:]
    softplus = jnp.maximum(logvar, 0.0) + jnp.log1p(jnp.exp(-jnp.abs(logvar)))
    std = 1e-6 + softplus
    z = mu + eps_ref[...] * std
    d1 = jnp.tanh(_dot(z, dw1_ref[...]) + db1_ref[...])
    d2 = elu(_dot(d1, dw2_ref[...]) + db2_ref[...])
    y = 1.0 / (1.0 + jnp.exp(-(_dot(d2, dw3_ref[...]) + db3_ref[...])))
    mu_ref[...] = mu
    std_ref[...] = std
    y_ref[...] = jnp.clip(y, 1e-8, 1.0 - 1e-8)


_head_call = pl.pallas_call(
    _head_body,
    in_specs=([pl.BlockSpec(memory_space=pltpu.MemorySpace.VMEM)] * 6
              + [pl.BlockSpec(memory_space=pltpu.MemorySpace.SMEM)]
              + [pl.BlockSpec(memory_space=pltpu.MemorySpace.VMEM)] * 13),
    out_shape=[
        jax.ShapeDtypeStruct((G, D_OUT), jnp.float32),
        jax.ShapeDtypeStruct((G, D_OUT), jnp.float32),
        jax.ShapeDtypeStruct((G, D), jnp.float32),
    ],
    scratch_shapes=[pltpu.VMEM((G, D), jnp.float32)],
)


def kernel(x, edge_index, batch, W_gcn, b_gcn, enc_W1, enc_b1, enc_W2, enc_b2,
           enc_W3, enc_b3, dec_W1, dec_b1, dec_W2, dec_b2, dec_W3, dec_b3):
    pad = EPAD - E
    src_p = jnp.concatenate(
        [edge_index[0], jnp.zeros((pad,), jnp.int32)]).reshape(EPAD // CHUNK, CHUNK)
    pad_dst = N + (jnp.arange(pad, dtype=jnp.int32) % (NPAD - N))
    dst_p = jnp.concatenate([edge_index[1], pad_dst]).reshape(EPAD // CHUNK, CHUNK)
    zeros1 = jnp.zeros((RPT,), jnp.float32)
    zeros2 = jnp.zeros((CHUNK, D), jnp.float32)

    hist = _deg_kernel(dst_p, zeros1)
    h2, dinv = _mm_call(x, W_gcn, hist.reshape(NC, NPAD, 1))
    p = _agg_kernel(h2, src_p, dst_p, zeros2)

    eps = jax.random.normal(jax.random.key(42), (G, D_OUT), dtype=jnp.float32)
    bnds = jnp.stack([batch[0::WIN], batch[WIN - 1::WIN]], axis=1)
    mu, std, y = _head_call(
        p, h2, dinv, b_gcn.reshape(1, D), batch.reshape(N, 1), batch.reshape(1, N),
        bnds,
        enc_W1, enc_b1.reshape(1, H_MLP), enc_W2, enc_b2.reshape(1, H_MLP),
        enc_W3, enc_b3.reshape(1, 2 * D_OUT), dec_W1, dec_b1.reshape(1, H_MLP),
        dec_W2, dec_b2.reshape(1, H_MLP), dec_W3, dec_b3.reshape(1, D), eps)
    return (mu, std, y)
